# baseline (speedup 1.0000x reference)
import jax
import jax.numpy as jnp
from jax.experimental import pallas as pl
from jax.experimental.pallas import tpu as pltpu

TOK = 512
NBUF = 4


def _gating_body(x_hbm, w_ref, b_ref, o_ref, xbuf, sem):
    nchunk = x_hbm.shape[0] // TOK
    w = w_ref[...].astype(jnp.bfloat16)
    bias = b_ref[...]

    def copy_in(i):
        pltpu.make_async_copy(
            x_hbm.at[pl.ds(i * TOK, TOK), :],
            xbuf.at[i % NBUF],
            sem.at[i % NBUF],
        ).start()

    for i in range(NBUF):
        copy_in(i)

    for i in range(nchunk):
        slot = i % NBUF
        pltpu.make_async_copy(
            x_hbm.at[pl.ds(i * TOK, TOK), :],
            xbuf.at[slot],
            sem.at[slot],
        ).wait()
        xh = xbuf[slot].astype(jnp.bfloat16)
        logits = jnp.dot(xh, w, preferred_element_type=jnp.float32) + bias
        m = jnp.max(logits, axis=-1, keepdims=True)
        e = jnp.exp(logits - m)
        o_ref[pl.ds(i * TOK, TOK), :] = e / jnp.sum(e, axis=-1, keepdims=True)
        if i + NBUF < nchunk:
            copy_in(i + NBUF)


def kernel(x, W, b):
    B, S, D = x.shape
    E = W.shape[1]
    N = B * S
    xf = x.reshape(N, D)
    b2 = b.reshape(1, E)

    out = pl.pallas_call(
        _gating_body,
        in_specs=[
            pl.BlockSpec(memory_space=pl.ANY),
            pl.BlockSpec(memory_space=pltpu.VMEM),
            pl.BlockSpec(memory_space=pltpu.VMEM),
        ],
        out_specs=pl.BlockSpec(memory_space=pltpu.VMEM),
        out_shape=jax.ShapeDtypeStruct((N, E), jnp.float32),
        scratch_shapes=[
            pltpu.VMEM((NBUF, TOK, D), jnp.float32),
            pltpu.SemaphoreType.DMA((NBUF,)),
        ],
    )(xf, W, b2)
    return out.reshape(B, S, E)
